# baseline (speedup 1.0000x reference)
import jax
import jax.numpy as jnp
from jax import lax
from jax.experimental import pallas as pl
from jax.experimental.pallas import tpu as pltpu
from jax.experimental.pallas import tpu_sc as plsc

_VOCAB = 1000000
_D = 64
_B = 4096
_N = 200
_TOT = _B * _N
_NW = 32
_PER_W = _TOT // _NW
_BBLK = 128
_NBUF = 4
_NTB = 2


def _bc16(x):
    return lax.broadcast(x, (16,))


def _sc_body(xflat_hbm, tab_hbm, pos_hbm, out_hbm, idx_v, pos_v, gidx_v,
             grows, tbuf, gsem, wsem):
    _IOTA = lax.iota(jnp.int32, 16)
    wid = lax.axis_index("s") * 2 + lax.axis_index("c")
    base = wid * _PER_W
    pltpu.sync_copy(xflat_hbm.at[pl.ds(base, _PER_W)], idx_v)
    pltpu.sync_copy(pos_hbm, pos_v)

    def prefetch(n, b):
        for g in range(8):
            iv = plsc.load_gather(idx_v, [(g * 16 + _IOTA) * _N + n])
            gidx_v[b, pl.ds(g * 16, 16)] = iv
        pltpu.async_copy(tab_hbm.at[gidx_v.at[b]], grows.at[b], gsem.at[b])

    for b in range(_NBUF):
        prefetch(b, b)

    def grp_body(gi, carry):
        for b in range(_NBUF):
            n = gi * _NBUF + b
            tb = b % _NTB
            pltpu.make_async_copy(
                tab_hbm.at[gidx_v.at[b]], grows.at[b], gsem.at[b]
            ).wait()

            @pl.when(n >= _NTB)
            def _():
                for tr in range(8):
                    pltpu.make_async_copy(
                        tbuf.at[tb, tr],
                        out_hbm.at[n - _NTB, tr, wid],
                        wsem.at[tb],
                    ).wait()

            pv = plsc.load_gather(pos_v, [_bc16(n)])
            gview = grows.at[b]
            for g in range(8):
                cvec = g * 16 + _IOTA

                def d_body(d, c2):
                    val = plsc.load_gather(gview, [cvec, _bc16(d)])
                    tbuf[tb, d // 8, d % 8, pl.ds(g * 16, 16)] = val + pv
                    return c2

                lax.fori_loop(0, _D, d_body, 0, unroll=8)

            for tr in range(8):
                pltpu.async_copy(
                    tbuf.at[tb, tr], out_hbm.at[n, tr, wid], wsem.at[tb]
                )

            @pl.when(gi < (_N // _NBUF) - 1)
            def _():
                prefetch(n + _NBUF, b)
        return carry

    lax.fori_loop(0, _N // _NBUF, grp_body, 0)

    for n in (_N - 2, _N - 1):
        tb = n % _NTB
        for tr in range(8):
            pltpu.make_async_copy(
                tbuf.at[tb, tr], out_hbm.at[n, tr, wid], wsem.at[tb]
            ).wait()


def kernel(X, table, pos):
    xflat = X.reshape(_TOT)
    tab0 = table.at[0].set(0.0)
    p1 = pos.astype(jnp.float32).reshape(_N)
    k = pl.kernel(
        _sc_body,
        mesh=plsc.VectorSubcoreMesh(core_axis_name="c", subcore_axis_name="s"),
        compiler_params=pltpu.CompilerParams(
            needs_layout_passes=False, use_tc_tiling_on_sc=False
        ),
        out_type=jax.ShapeDtypeStruct((_N, 8, _NW, 8, _BBLK), jnp.float32),
        scratch_types=[
            pltpu.VMEM((_PER_W,), jnp.int32),
            pltpu.VMEM((_N,), jnp.float32),
            pltpu.VMEM((_NBUF, _BBLK), jnp.int32),
            pltpu.VMEM((_NBUF, _BBLK, _D), jnp.float32),
            pltpu.VMEM((_NTB, 8, 8, _BBLK), jnp.float32),
            pltpu.SemaphoreType.DMA((_NBUF,)),
            pltpu.SemaphoreType.DMA((_NTB,)),
        ],
    )
    out5 = k(xflat, tab0, p1)
    return out5.transpose((2, 4, 0, 1, 3)).reshape(_B, _N, _D)


# speedup vs baseline: 1.1402x; 1.1402x over previous
import jax
import jax.numpy as jnp
from jax import lax
from jax.experimental import pallas as pl
from jax.experimental.pallas import tpu as pltpu
from jax.experimental.pallas import tpu_sc as plsc

_VOCAB = 1000000
_D = 64
_B = 4096
_N = 200
_TOT = _B * _N
_NW = 32
_PER_W = _TOT // _NW
_BBLK = 128
_NBUF = 4
_NTB = 2


def _bc16(x):
    return lax.broadcast(x, (16,))


def _sc_body(xflat_hbm, tab_hbm, pos_hbm, out_hbm, idx_v, pos_v, gidx_v,
             grows, tbuf, gsem, wsem):
    _IOTA = lax.iota(jnp.int32, 16)
    wid = lax.axis_index("s") * 2 + lax.axis_index("c")
    base = wid * _PER_W
    pltpu.sync_copy(xflat_hbm.at[pl.ds(base, _PER_W)], idx_v)
    pltpu.sync_copy(pos_hbm, pos_v)

    def prefetch(n, b):
        for g in range(8):
            iv = plsc.load_gather(idx_v, [(g * 16 + _IOTA) * _N + n])
            gidx_v[b, pl.ds(g * 16, 16)] = iv
        pltpu.async_copy(tab_hbm.at[gidx_v.at[b]], grows.at[b], gsem.at[b])

    for b in range(_NBUF):
        prefetch(b, b)

    def grp_body(gi, carry):
        for b in range(_NBUF):
            n = gi * _NBUF + b
            tb = b % _NTB
            pltpu.make_async_copy(
                tab_hbm.at[gidx_v.at[b]], grows.at[b], gsem.at[b]
            ).wait()

            out_v = out_hbm

            @pl.when(n >= _NTB)
            def _():
                for tr in range(8):
                    pltpu.make_async_copy(
                        tbuf.at[tb, pl.ds(tr * 8 * _BBLK, 8 * _BBLK)],
                        out_v.at[n - _NTB, tr, wid],
                        wsem.at[tb],
                    ).wait()

            pv = plsc.load_gather(pos_v, [_bc16(n)])
            tview = tbuf.at[tb]
            cvecs = [(dg * 16 + _IOTA) * _BBLK for dg in range(4)]

            def r_body(r, c2):
                rb = _bc16(r)
                for dg in range(4):
                    val = grows[b, r, pl.ds(dg * 16, 16)]
                    plsc.store_scatter(tview, [cvecs[dg] + rb], val + pv)
                return c2

            lax.fori_loop(0, _BBLK, r_body, 0, unroll=4)

            for tr in range(8):
                pltpu.async_copy(
                    tbuf.at[tb, pl.ds(tr * 8 * _BBLK, 8 * _BBLK)],
                    out_v.at[n, tr, wid],
                    wsem.at[tb],
                )

            @pl.when(gi < (_N // _NBUF) - 1)
            def _():
                prefetch(n + _NBUF, b)
        return carry

    lax.fori_loop(0, _N // _NBUF, grp_body, 0)

    out_v = out_hbm
    for n in (_N - 2, _N - 1):
        tb = n % _NTB
        for tr in range(8):
            pltpu.make_async_copy(
                tbuf.at[tb, pl.ds(tr * 8 * _BBLK, 8 * _BBLK)],
                out_v.at[n, tr, wid],
                wsem.at[tb],
            ).wait()


def kernel(X, table, pos):
    xflat = X.reshape(_TOT)
    tab0 = table.at[0].set(0.0)
    p1 = pos.astype(jnp.float32).reshape(_N)
    k = pl.kernel(
        _sc_body,
        mesh=plsc.VectorSubcoreMesh(core_axis_name="c", subcore_axis_name="s"),
        compiler_params=pltpu.CompilerParams(
            needs_layout_passes=False, use_tc_tiling_on_sc=False
        ),
        out_type=jax.ShapeDtypeStruct((_N, 8, _NW, 8 * _BBLK), jnp.float32),
        scratch_types=[
            pltpu.VMEM((_PER_W,), jnp.int32),
            pltpu.VMEM((_N,), jnp.float32),
            pltpu.VMEM((_NBUF, _BBLK), jnp.int32),
            pltpu.VMEM((_NBUF, _BBLK, _D), jnp.float32),
            pltpu.VMEM((_NTB, 8 * 8 * _BBLK), jnp.float32),
            pltpu.SemaphoreType.DMA((_NBUF,)),
            pltpu.SemaphoreType.DMA((_NTB,)),
        ],
    )
    out4 = k(xflat, tab0, p1)
    out5 = out4.reshape(_N, 8, _NW, 8, _BBLK)
    return out5.transpose((2, 4, 0, 1, 3)).reshape(_B, _N, _D)


# speedup vs baseline: 1.7636x; 1.5467x over previous
import jax
import jax.numpy as jnp
from jax import lax
from jax.experimental import pallas as pl
from jax.experimental.pallas import tpu as pltpu
from jax.experimental.pallas import tpu_sc as plsc

_VOCAB = 1000000
_D = 64
_B = 4096
_N = 200
_TOT = _B * _N
_NW = 32
_PER_W = _TOT // _NW
_BBLK = 128
_NBUF = 4
_NTB = 2
_TSTR = 129


def _bc16(x):
    return lax.broadcast(x, (16,))


def _sc_body(xflat_hbm, tab_hbm, pos_hbm, out_hbm, idx_v, pos_v, gidx_v,
             grows, tbuf, gsem, wsem):
    _IOTA = lax.iota(jnp.int32, 16)
    wid = lax.axis_index("s") * 2 + lax.axis_index("c")
    base = wid * _PER_W
    pltpu.sync_copy(xflat_hbm.at[pl.ds(base, _PER_W)], idx_v)
    pltpu.sync_copy(pos_hbm, pos_v)

    def prefetch(n, b):
        for g in range(8):
            iv = plsc.load_gather(idx_v, [(g * 16 + _IOTA) * _N + n])
            gidx_v[b, pl.ds(g * 16, 16)] = iv
        pltpu.async_copy(tab_hbm.at[gidx_v.at[b]], grows.at[b], gsem.at[b])

    for b in range(_NBUF):
        prefetch(b, b)

    def grp_body(gi, carry):
        for b in range(_NBUF):
            n = gi * _NBUF + b
            tb = b % _NTB
            pltpu.make_async_copy(
                tab_hbm.at[gidx_v.at[b]], grows.at[b], gsem.at[b]
            ).wait()

            @pl.when(n >= _NTB)
            def _():
                for tr in range(8):
                    pltpu.make_async_copy(
                        tbuf.at[tb, tr, :, pl.ds(0, _BBLK)],
                        out_hbm.at[n - _NTB, tr, wid],
                        wsem.at[tb],
                    ).wait()

            pv = plsc.load_gather(pos_v, [_bc16(n)])
            tview = tbuf.at[tb]
            trvs = [2 * dg + _IOTA // 8 for dg in range(4)]
            rrv = _IOTA % 8

            def r_body(r, c2):
                rb = _bc16(r)
                for dg in range(4):
                    val = grows[b, r, pl.ds(dg * 16, 16)]
                    plsc.store_scatter(tview, [trvs[dg], rrv, rb], val + pv)
                return c2

            lax.fori_loop(0, _BBLK, r_body, 0, unroll=4)

            for tr in range(8):
                pltpu.async_copy(
                    tbuf.at[tb, tr, :, pl.ds(0, _BBLK)],
                    out_hbm.at[n, tr, wid],
                    wsem.at[tb],
                )

            @pl.when(gi < (_N // _NBUF) - 1)
            def _():
                prefetch(n + _NBUF, b)
        return carry

    lax.fori_loop(0, _N // _NBUF, grp_body, 0)

    for n in (_N - 2, _N - 1):
        tb = n % _NTB
        for tr in range(8):
            pltpu.make_async_copy(
                tbuf.at[tb, tr, :, pl.ds(0, _BBLK)],
                out_hbm.at[n, tr, wid],
                wsem.at[tb],
            ).wait()


def kernel(X, table, pos):
    xflat = X.reshape(_TOT)
    tab0 = table.at[0].set(0.0)
    p1 = pos.astype(jnp.float32).reshape(_N)
    k = pl.kernel(
        _sc_body,
        mesh=plsc.VectorSubcoreMesh(core_axis_name="c", subcore_axis_name="s"),
        compiler_params=pltpu.CompilerParams(
            needs_layout_passes=False, use_tc_tiling_on_sc=False
        ),
        out_type=jax.ShapeDtypeStruct((_N, 8, _NW, 8, _BBLK), jnp.float32),
        scratch_types=[
            pltpu.VMEM((_PER_W,), jnp.int32),
            pltpu.VMEM((_N,), jnp.float32),
            pltpu.VMEM((_NBUF, _BBLK), jnp.int32),
            pltpu.VMEM((_NBUF, _BBLK, _D), jnp.float32),
            pltpu.VMEM((_NTB, 8, 8, _TSTR), jnp.float32),
            pltpu.SemaphoreType.DMA((_NBUF,)),
            pltpu.SemaphoreType.DMA((_NTB,)),
        ],
    )
    out5 = k(xflat, tab0, p1)
    return out5.transpose((2, 4, 0, 1, 3)).reshape(_B, _N, _D)
